# baseline (speedup 1.0000x reference)
import functools

import jax
import jax.numpy as jnp
import numpy as np
from jax.experimental import pallas as pl
from jax.experimental.pallas import tpu as pltpu

N = 10000
NP = 10240
H = 256
F2 = 512
BR = 512
BK = 1024
Q_VAL = 0.25


def _prep_weights(edge_index, edge_weight, num_nodes):
    e = edge_index.shape[1]
    src0 = edge_index[0].astype(jnp.int32)
    dst0 = edge_index[1].astype(jnp.int32)
    self_mask = src0 != dst0
    ei2_src = jnp.concatenate([src0, dst0])
    ei2_dst = jnp.concatenate([dst0, src0])
    keep = jnp.concatenate([self_mask, self_mask])
    lin = ei2_src * num_nodes + ei2_dst
    sentinel = num_nodes * num_nodes
    lin = jnp.where(keep, lin, sentinel)
    ew = edge_weight
    sym0 = jnp.where(keep, jnp.concatenate([ew, ew]), 0.0)
    theta0 = jnp.where(keep, jnp.concatenate([ew, -ew]), 0.0)
    lin_s, sym_attr, theta_attr = jax.lax.sort((lin, sym0, theta0), num_keys=1)
    new_seg = jnp.concatenate([jnp.ones((1,), dtype=bool), lin_s[1:] != lin_s[:-1]])
    seg_ids = (jnp.cumsum(new_seg.astype(jnp.int32)) - 1).astype(jnp.int32)
    n_uniq = 2 * e
    lin_u = jax.ops.segment_sum(jnp.where(new_seg, lin_s, 0), seg_ids, num_segments=n_uniq)
    row = jnp.clip(lin_u // num_nodes, 0, num_nodes - 1).astype(jnp.int32)
    col = jnp.clip(lin_u % num_nodes, 0, num_nodes - 1).astype(jnp.int32)
    ew_sym = jax.ops.segment_sum(sym_attr, seg_ids, num_segments=n_uniq) / 2.0
    theta = jax.ops.segment_sum(theta_attr, seg_ids, num_segments=n_uniq)
    deg = jax.ops.segment_sum(ew_sym, row, num_segments=num_nodes)
    safe = jnp.where(deg > 0, deg, 1.0)
    dinv = jnp.where(deg > 0, 1.0 / jnp.sqrt(safe), 0.0)
    wnorm = dinv[row] * ew_sym * dinv[col]
    ang = 2.0 * np.pi * Q_VAL * theta
    wr = -wnorm * jnp.cos(ang)
    wi = -wnorm * jnp.sin(ang)
    return row, col, wr, wi


def _prop_body(mr_ref, mi_ref, xa_ref, xb_ref, pr_ref, pi_ref, accr, acci):
    j = pl.program_id(1)

    @pl.when(j == 0)
    def _():
        accr[...] = jnp.zeros_like(accr)
        acci[...] = jnp.zeros_like(acci)

    xa = xa_ref[pl.ds(j * BK, BK), :]
    xb = xb_ref[pl.ds(j * BK, BK), :]
    accr[...] += jnp.dot(mr_ref[...], xa, preferred_element_type=jnp.float32,
                         precision=jax.lax.Precision.HIGHEST)
    acci[...] += jnp.dot(mi_ref[...], xb, preferred_element_type=jnp.float32,
                         precision=jax.lax.Precision.HIGHEST)

    @pl.when(j == pl.num_programs(1) - 1)
    def _():
        pr_ref[...] = accr[...]
        pi_ref[...] = acci[...]


def _prop(Mr, Mi, Xa, Xb):
    return pl.pallas_call(
        _prop_body,
        grid=(NP // BR, NP // BK),
        in_specs=[
            pl.BlockSpec((BR, BK), lambda i, j: (i, j)),
            pl.BlockSpec((BR, BK), lambda i, j: (i, j)),
            pl.BlockSpec((NP, F2), lambda i, j: (0, 0)),
            pl.BlockSpec((NP, F2), lambda i, j: (0, 0)),
        ],
        out_specs=[
            pl.BlockSpec((BR, F2), lambda i, j: (i, 0)),
            pl.BlockSpec((BR, F2), lambda i, j: (i, 0)),
        ],
        out_shape=[
            jax.ShapeDtypeStruct((NP, F2), jnp.float32),
            jax.ShapeDtypeStruct((NP, F2), jnp.float32),
        ],
        scratch_shapes=[
            pltpu.VMEM((BR, F2), jnp.float32),
            pltpu.VMEM((BR, F2), jnp.float32),
        ],
    )(Mr, Mi, Xa, Xb)


def _tracks(x_ref, pr_ref, pi_ref, qr_ref, qi_ref, w_ref, b_ref):
    x = x_ref[...]
    xr = x[:, :H]
    xi = x[:, H:]
    pr = pr_ref[...]
    pi = pi_ref[...]
    qr = qr_ref[...]
    qi = qi_ref[...]
    t10 = pr[:, :H]
    t11 = -pi[:, H:]
    t12 = -pi[:, :H]
    t13 = pr[:, H:]
    t20 = 2.0 * qr[:, :H] - xr
    t21 = 2.0 * qi[:, H:] - xi
    t22 = 2.0 * qi[:, :H] - xr
    t23 = 2.0 * qr[:, H:] - xi
    w0 = w_ref[0]
    w1 = w_ref[1]
    w2 = w_ref[2]

    def d(a, wk):
        return jnp.dot(a, wk, preferred_element_type=jnp.float32)

    xrw0 = d(xr, w0)
    xiw0 = d(xi, w0)
    out0 = xrw0 + d(t10, w1) + d(t20, w2)
    out1 = xiw0 + d(t11, w1) + d(t21, w2)
    out2 = xrw0 + d(t12, w1) + d(t22, w2)
    out3 = xiw0 + d(t13, w1) + d(t23, w2)
    b = b_ref[...]
    yr = out0 - out1 + b
    yi = out2 + out3 + b
    m = (yr >= 0).astype(jnp.float32)
    return m * yr, m * yi


def _combine_body(x_ref, pr_ref, pi_ref, qr_ref, qi_ref, w_ref, b_ref, y_ref):
    yr, yi = _tracks(x_ref, pr_ref, pi_ref, qr_ref, qi_ref, w_ref, b_ref)
    y_ref[...] = jnp.concatenate([yr, yi], axis=1)


def _combine2_body(x_ref, pr_ref, pi_ref, qr_ref, qi_ref, w_ref, b_ref, a_ref,
                   y_ref, g_ref):
    yr, yi = _tracks(x_ref, pr_ref, pi_ref, qr_ref, qi_ref, w_ref, b_ref)
    y = jnp.concatenate([yr, yi], axis=1)
    y_ref[...] = y
    g_ref[...] = jnp.dot(y, a_ref[...], preferred_element_type=jnp.float32,
                         precision=jax.lax.Precision.HIGHEST)


def _combine(X, Pr, Pi, Qr, Qi, Wc, b):
    return pl.pallas_call(
        _combine_body,
        grid=(NP // BR,),
        in_specs=[
            pl.BlockSpec((BR, F2), lambda i: (i, 0)),
            pl.BlockSpec((BR, F2), lambda i: (i, 0)),
            pl.BlockSpec((BR, F2), lambda i: (i, 0)),
            pl.BlockSpec((BR, F2), lambda i: (i, 0)),
            pl.BlockSpec((BR, F2), lambda i: (i, 0)),
            pl.BlockSpec((3, H, H), lambda i: (0, 0, 0)),
            pl.BlockSpec((H,), lambda i: (0,)),
        ],
        out_specs=pl.BlockSpec((BR, F2), lambda i: (i, 0)),
        out_shape=jax.ShapeDtypeStruct((NP, F2), jnp.float32),
    )(X, Pr, Pi, Qr, Qi, Wc, b)


def _combine2(X, Pr, Pi, Qr, Qi, Wc, b, Acat):
    return pl.pallas_call(
        _combine2_body,
        grid=(NP // BR,),
        in_specs=[
            pl.BlockSpec((BR, F2), lambda i: (i, 0)),
            pl.BlockSpec((BR, F2), lambda i: (i, 0)),
            pl.BlockSpec((BR, F2), lambda i: (i, 0)),
            pl.BlockSpec((BR, F2), lambda i: (i, 0)),
            pl.BlockSpec((BR, F2), lambda i: (i, 0)),
            pl.BlockSpec((3, H, H), lambda i: (0, 0, 0)),
            pl.BlockSpec((H,), lambda i: (0,)),
            pl.BlockSpec((F2, 4), lambda i: (0, 0)),
        ],
        out_specs=[
            pl.BlockSpec((BR, F2), lambda i: (i, 0)),
            pl.BlockSpec((BR, 4), lambda i: (i, 0)),
        ],
        out_shape=[
            jax.ShapeDtypeStruct((NP, F2), jnp.float32),
            jax.ShapeDtypeStruct((NP, 4), jnp.float32),
        ],
    )(X, Pr, Pi, Qr, Qi, Wc, b, Acat)


def _lsm_body(z_ref, o_ref):
    z = z_ref[...]
    m = jnp.max(z, axis=1, keepdims=True)
    s = z - m
    lse = jnp.log(jnp.sum(jnp.exp(s), axis=1, keepdims=True))
    o_ref[...] = s - lse


def _lsm(z):
    n = z.shape[0]
    return pl.pallas_call(
        _lsm_body,
        grid=(10,),
        in_specs=[pl.BlockSpec((n // 10, 2), lambda i: (i, 0))],
        out_specs=pl.BlockSpec((n // 10, 2), lambda i: (i, 0)),
        out_shape=jax.ShapeDtypeStruct((n, 2), jnp.float32),
    )(z)


def kernel(real, imag, edge_index, query_edges, edge_weight, W1, b1, W2, b2, lin_W, lin_b):
    num_nodes = real.shape[0]
    row, col, wr, wi = _prep_weights(edge_index, edge_weight, num_nodes)

    Mr = jnp.zeros((NP, NP), jnp.float32).at[row, col].add(wr)
    Mi = jnp.zeros((NP, NP), jnp.float32).at[row, col].add(wi)

    X0 = jnp.zeros((NP, F2), jnp.float32)
    X0 = X0.at[:N, :H].set(real).at[:N, H:].set(imag)

    A0, A1, A2, A3 = lin_W[:H], lin_W[H:2 * H], lin_W[2 * H:3 * H], lin_W[3 * H:]
    Acat = jnp.concatenate(
        [jnp.concatenate([A0, A1], axis=1), jnp.concatenate([A2, A3], axis=1)], axis=0)

    Pr, Pi = _prop(Mr, Mi, X0, X0)
    Qr, Qi = _prop(Mr, Mi, Pr, Pi)
    Y1 = _combine(X0, Pr, Pi, Qr, Qi, W1, b1)

    Pr2, Pi2 = _prop(Mr, Mi, Y1, Y1)
    Qr2, Qi2 = _prop(Mr, Mi, Pr2, Pi2)
    Y2, G = _combine2(Y1, Pr2, Pi2, Qr2, Qi2, W2, b2, Acat)

    embedding = Y2[:N]
    q0 = query_edges[:, 0]
    q1 = query_edges[:, 1]
    z = G[q0, :2] + G[q1, 2:] + lin_b
    x = _lsm(z)
    return embedding, x


# speedup vs baseline: 1.3193x; 1.3193x over previous
import functools

import jax
import jax.numpy as jnp
import numpy as np
from jax.experimental import pallas as pl
from jax.experimental.pallas import tpu as pltpu

N = 10000
NP = 10240
H = 256
F2 = 512
BR = 512
BK = 1024
Q_VAL = 0.25


def _prep_weights(edge_index, edge_weight, num_nodes):
    e = edge_index.shape[1]
    src0 = edge_index[0].astype(jnp.int32)
    dst0 = edge_index[1].astype(jnp.int32)
    self_mask = src0 != dst0
    ei2_src = jnp.concatenate([src0, dst0])
    ei2_dst = jnp.concatenate([dst0, src0])
    keep = jnp.concatenate([self_mask, self_mask])
    lin = ei2_src * num_nodes + ei2_dst
    sentinel = num_nodes * num_nodes
    lin = jnp.where(keep, lin, sentinel)
    dirbit = jnp.concatenate([jnp.zeros((e,), jnp.int32), jnp.ones((e,), jnp.int32)])
    key = lin * 2 + dirbit
    ew2 = jnp.concatenate([edge_weight, edge_weight])
    key_s, ew_s = jax.lax.sort((key, ew2), num_keys=1)
    lin_s = key_s >> 1
    live = key_s < sentinel * 2
    sym_attr = jnp.where(live, ew_s, 0.0)
    theta_attr = jnp.where((key_s & 1) == 1, -sym_attr, sym_attr)
    new_seg = jnp.concatenate([jnp.ones((1,), dtype=bool), lin_s[1:] != lin_s[:-1]])
    seg_ids = (jnp.cumsum(new_seg.astype(jnp.int32)) - 1).astype(jnp.int32)
    n_uniq = 2 * e
    lin_u = jax.ops.segment_sum(jnp.where(new_seg, lin_s, 0), seg_ids, num_segments=n_uniq)
    row = jnp.clip(lin_u // num_nodes, 0, num_nodes - 1).astype(jnp.int32)
    col = jnp.clip(lin_u % num_nodes, 0, num_nodes - 1).astype(jnp.int32)
    ew_sym = jax.ops.segment_sum(sym_attr, seg_ids, num_segments=n_uniq) / 2.0
    theta = jax.ops.segment_sum(theta_attr, seg_ids, num_segments=n_uniq)
    deg = jax.ops.segment_sum(ew_sym, row, num_segments=num_nodes)
    safe = jnp.where(deg > 0, deg, 1.0)
    dinv = jnp.where(deg > 0, 1.0 / jnp.sqrt(safe), 0.0)
    ang = 2.0 * np.pi * Q_VAL * theta
    ws_r = ew_sym * jnp.cos(ang)
    ws_i = ew_sym * jnp.sin(ang)
    return row, col, ws_r, ws_i, dinv, -(dinv * dinv)


def _prop_body(mr_ref, mi_ref, xa_ref, xb_ref, pr_ref, pi_ref, accr, acci):
    j = pl.program_id(1)

    @pl.when(j == 0)
    def _():
        accr[...] = jnp.zeros_like(accr)
        acci[...] = jnp.zeros_like(acci)

    xa = xa_ref[pl.ds(j * BK, BK), :]
    xb = xb_ref[pl.ds(j * BK, BK), :]
    accr[...] += jnp.dot(mr_ref[...], xa, preferred_element_type=jnp.float32,
                         precision=jax.lax.Precision.HIGHEST)
    acci[...] += jnp.dot(mi_ref[...], xb, preferred_element_type=jnp.float32,
                         precision=jax.lax.Precision.HIGHEST)

    @pl.when(j == pl.num_programs(1) - 1)
    def _():
        pr_ref[...] = accr[...]
        pi_ref[...] = acci[...]


def _prop(Sr, Si, Xa, Xb):
    return pl.pallas_call(
        _prop_body,
        grid=(NP // BR, NP // BK),
        in_specs=[
            pl.BlockSpec((BR, BK), lambda i, j: (i, j)),
            pl.BlockSpec((BR, BK), lambda i, j: (i, j)),
            pl.BlockSpec((NP, F2), lambda i, j: (0, 0)),
            pl.BlockSpec((NP, F2), lambda i, j: (0, 0)),
        ],
        out_specs=[
            pl.BlockSpec((BR, F2), lambda i, j: (i, 0)),
            pl.BlockSpec((BR, F2), lambda i, j: (i, 0)),
        ],
        out_shape=[
            jax.ShapeDtypeStruct((NP, F2), jnp.float32),
            jax.ShapeDtypeStruct((NP, F2), jnp.float32),
        ],
        scratch_shapes=[
            pltpu.VMEM((BR, F2), jnp.float32),
            pltpu.VMEM((BR, F2), jnp.float32),
        ],
    )(Sr, Si, Xa, Xb)


def _tracks(x_ref, pr_ref, pi_ref, qr_ref, qi_ref, w_ref, b_ref, dv_ref):
    x = x_ref[...]
    xr = x[:, :H]
    xi = x[:, H:]
    dv = dv_ref[...]
    ps = pr_ref[...]
    psi = pi_ref[...]
    qs = qr_ref[...]
    qsi = qi_ref[...]
    t10 = -dv * ps[:, :H]
    t11 = dv * psi[:, H:]
    t12 = dv * psi[:, :H]
    t13 = -dv * ps[:, H:]
    dv2 = -2.0 * dv
    t20 = dv2 * qs[:, :H] - xr
    t21 = dv2 * qsi[:, H:] - xi
    t22 = dv2 * qsi[:, :H] - xr
    t23 = dv2 * qs[:, H:] - xi
    w0 = w_ref[0]
    w1 = w_ref[1]
    w2 = w_ref[2]

    def d(a, wk):
        return jnp.dot(a, wk, preferred_element_type=jnp.float32)

    xrw0 = d(xr, w0)
    xiw0 = d(xi, w0)
    out0 = xrw0 + d(t10, w1) + d(t20, w2)
    out1 = xiw0 + d(t11, w1) + d(t21, w2)
    out2 = xrw0 + d(t12, w1) + d(t22, w2)
    out3 = xiw0 + d(t13, w1) + d(t23, w2)
    b = b_ref[...]
    yr = out0 - out1 + b
    yi = out2 + out3 + b
    m = (yr >= 0).astype(jnp.float32)
    return m * yr, m * yi


def _combine_body(x_ref, pr_ref, pi_ref, qr_ref, qi_ref, w_ref, b_ref, dv_ref, y_ref):
    yr, yi = _tracks(x_ref, pr_ref, pi_ref, qr_ref, qi_ref, w_ref, b_ref, dv_ref)
    y_ref[...] = jnp.concatenate([yr, yi], axis=1)


def _combine2_body(x_ref, pr_ref, pi_ref, qr_ref, qi_ref, w_ref, b_ref, a_ref,
                    dv_ref, y_ref, g_ref):
    yr, yi = _tracks(x_ref, pr_ref, pi_ref, qr_ref, qi_ref, w_ref, b_ref, dv_ref)
    y = jnp.concatenate([yr, yi], axis=1)
    y_ref[...] = y
    g_ref[...] = jnp.dot(y, a_ref[...], preferred_element_type=jnp.float32,
                         precision=jax.lax.Precision.HIGHEST)


def _combine(X, Pr, Pi, Qr, Qi, Wc, b, dv):
    return pl.pallas_call(
        _combine_body,
        grid=(NP // BR,),
        in_specs=[
            pl.BlockSpec((BR, F2), lambda i: (i, 0)),
            pl.BlockSpec((BR, F2), lambda i: (i, 0)),
            pl.BlockSpec((BR, F2), lambda i: (i, 0)),
            pl.BlockSpec((BR, F2), lambda i: (i, 0)),
            pl.BlockSpec((BR, F2), lambda i: (i, 0)),
            pl.BlockSpec((3, H, H), lambda i: (0, 0, 0)),
            pl.BlockSpec((H,), lambda i: (0,)),
            pl.BlockSpec((BR, 1), lambda i: (i, 0)),
        ],
        out_specs=pl.BlockSpec((BR, F2), lambda i: (i, 0)),
        out_shape=jax.ShapeDtypeStruct((NP, F2), jnp.float32),
    )(X, Pr, Pi, Qr, Qi, Wc, b, dv)


def _combine2(X, Pr, Pi, Qr, Qi, Wc, b, Acat, dv):
    return pl.pallas_call(
        _combine2_body,
        grid=(NP // BR,),
        in_specs=[
            pl.BlockSpec((BR, F2), lambda i: (i, 0)),
            pl.BlockSpec((BR, F2), lambda i: (i, 0)),
            pl.BlockSpec((BR, F2), lambda i: (i, 0)),
            pl.BlockSpec((BR, F2), lambda i: (i, 0)),
            pl.BlockSpec((BR, F2), lambda i: (i, 0)),
            pl.BlockSpec((3, H, H), lambda i: (0, 0, 0)),
            pl.BlockSpec((H,), lambda i: (0,)),
            pl.BlockSpec((F2, 4), lambda i: (0, 0)),
            pl.BlockSpec((BR, 1), lambda i: (i, 0)),
        ],
        out_specs=[
            pl.BlockSpec((BR, F2), lambda i: (i, 0)),
            pl.BlockSpec((BR, 4), lambda i: (i, 0)),
        ],
        out_shape=[
            jax.ShapeDtypeStruct((NP, F2), jnp.float32),
            jax.ShapeDtypeStruct((NP, 4), jnp.float32),
        ],
    )(X, Pr, Pi, Qr, Qi, Wc, b, Acat, dv)


def _lsm_body(z_ref, o_ref):
    z = z_ref[...]
    m = jnp.max(z, axis=1, keepdims=True)
    s = z - m
    lse = jnp.log(jnp.sum(jnp.exp(s), axis=1, keepdims=True))
    o_ref[...] = s - lse


def _lsm(z):
    n = z.shape[0]
    return pl.pallas_call(
        _lsm_body,
        grid=(10,),
        in_specs=[pl.BlockSpec((n // 10, 2), lambda i: (i, 0))],
        out_specs=pl.BlockSpec((n // 10, 2), lambda i: (i, 0)),
        out_shape=jax.ShapeDtypeStruct((n, 2), jnp.float32),
    )(z)


def kernel(real, imag, edge_index, query_edges, edge_weight, W1, b1, W2, b2, lin_W, lin_b):
    num_nodes = real.shape[0]
    row, col, ws_r, ws_i, dinv, sc2 = _prep_weights(edge_index, edge_weight, num_nodes)

    Sr = jnp.zeros((NP, NP), jnp.float32).at[row, col].add(ws_r)
    Si = jnp.zeros((NP, NP), jnp.float32).at[row, col].add(ws_i)

    X0 = jnp.zeros((NP, F2), jnp.float32)
    X0 = X0.at[:N, :H].set(real).at[:N, H:].set(imag)
    dv = jnp.zeros((NP, 1), jnp.float32).at[:N, 0].set(dinv)
    sc2v = jnp.zeros((NP, 1), jnp.float32).at[:N, 0].set(sc2)

    A0, A1, A2, A3 = lin_W[:H], lin_W[H:2 * H], lin_W[2 * H:3 * H], lin_W[3 * H:]
    Acat = jnp.concatenate(
        [jnp.concatenate([A0, A1], axis=1), jnp.concatenate([A2, A3], axis=1)], axis=0)

    Z0 = X0 * dv
    Ps, Psi = _prop(Sr, Si, Z0, Z0)
    Qs, Qsi = _prop(Sr, Si, Ps * sc2v, Psi * sc2v)
    Y1 = _combine(X0, Ps, Psi, Qs, Qsi, W1, b1, dv)

    Z1 = Y1 * dv
    Ps2, Psi2 = _prop(Sr, Si, Z1, Z1)
    Qs2, Qsi2 = _prop(Sr, Si, Ps2 * sc2v, Psi2 * sc2v)
    Y2, G = _combine2(Y1, Ps2, Psi2, Qs2, Qsi2, W2, b2, Acat, dv)

    embedding = Y2[:N]
    q0 = query_edges[:, 0]
    q1 = query_edges[:, 1]
    z = G[q0, :2] + G[q1, 2:] + lin_b
    x = _lsm(z)
    return embedding, x
